# baseline (speedup 1.0000x reference)
import functools

import jax
import jax.numpy as jnp
from jax import lax
from jax.experimental import pallas as pl
from jax.experimental.pallas import tpu as pltpu
from jax.experimental.pallas import tpu_sc as plsc

LANES = 16
CHUNK = 128
NSC = 2
NTILES = 16
NW = NSC * NTILES


def _round_up(v, m):
    return (v + m - 1) // m * m


def _sc_mesh():
    return plsc.VectorSubcoreMesh(core_axis_name="c", subcore_axis_name="s")


def _make_deg_kernel(ep, n_pad):
    nch = ep // NW // CHUNK
    rpt = n_pad // NTILES

    @functools.partial(
        pl.kernel,
        out_type=jax.ShapeDtypeStruct((NSC, n_pad), jnp.float32),
        mesh=_sc_mesh(),
        scratch_types=[
            pltpu.VMEM((nch, CHUNK), jnp.int32),
            pltpu.VMEM((nch, CHUNK), jnp.float32),
            pltpu.VMEM((rpt,), jnp.float32),
            pltpu.VMEM_SHARED((n_pad,), jnp.float32),
        ],
    )
    def deg_kernel(dst_hbm, w_hbm, out_hbm, dst_v, w_v, zbuf, acc_sh):
        c = lax.axis_index("c")
        s = lax.axis_index("s")
        wid = c * NTILES + s

        def zero_body(i, _):
            zbuf[pl.ds(i * LANES, LANES)] = jnp.zeros((LANES,), jnp.float32)
            return 0
        lax.fori_loop(0, rpt // LANES, zero_body, 0)
        pltpu.sync_copy(zbuf, acc_sh.at[pl.ds(s * rpt, rpt)])
        pltpu.sync_copy(dst_hbm.at[pl.ds(wid * nch, nch)], dst_v)
        pltpu.sync_copy(w_hbm.at[pl.ds(wid * nch, nch)], w_v)
        plsc.subcore_barrier()

        def chunk_body(j, _):
            pltpu.sync_copy(w_v.at[j], acc_sh.at[dst_v.at[j]], add=True)
            return 0
        lax.fori_loop(0, nch, chunk_body, 0)
        plsc.subcore_barrier()
        pltpu.sync_copy(acc_sh.at[pl.ds(s * rpt, rpt)],
                        out_hbm.at[c, pl.ds(s * rpt, rpt)])

    return deg_kernel


NSTAGE = 16
AGG_SHARE0 = 0.72


def _make_agg_kernel(ep, n_pad, d, share0=0.5):
    tot = ep // NTILES // CHUNK
    nch0 = int(round(tot * share0 / NSTAGE)) * NSTAGE
    nch1 = tot - nch0
    rpt = n_pad // NTILES
    vregs = d // LANES

    @functools.partial(
        pl.kernel,
        out_type=jax.ShapeDtypeStruct((NSC, n_pad, d), jnp.float32),
        mesh=_sc_mesh(),
        scratch_types=[
            pltpu.VMEM((NSTAGE, CHUNK), jnp.int32),
            pltpu.VMEM((NSTAGE, CHUNK), jnp.int32),
            pltpu.VMEM((NSTAGE, CHUNK), jnp.float32),
            pltpu.VMEM((CHUNK, d), jnp.float32),
            pltpu.VMEM((CHUNK, d), jnp.float32),
            pltpu.VMEM_SHARED((n_pad, d), jnp.float32),
            pltpu.SemaphoreType.DMA,
            pltpu.SemaphoreType.DMA,
        ],
        compiler_params=pltpu.CompilerParams(use_tc_tiling_on_sc=False),
    )
    def agg_kernel(tbl_hbm, src_hbm, dst_hbm, w_hbm, out_hbm,
                   src_v, dst_v, w_v, rows0, rows1, acc_sh, sem0, sem1):
        c = lax.axis_index("c")
        s = lax.axis_index("s")
        wid = c * NTILES + s

        def zero_body(i, _):
            r = i // vregs
            g = i % vregs
            rows0[r, pl.ds(g * LANES, LANES)] = jnp.zeros((LANES,), jnp.float32)
            return 0
        lax.fori_loop(0, CHUNK * vregs, zero_body, 0)
        for k in range(rpt // CHUNK):
            pltpu.sync_copy(rows0, acc_sh.at[pl.ds(s * rpt + k * CHUNK, CHUNK)])
        plsc.subcore_barrier()

        def scale(rows, j):
            def scale_body(g, _):
                wvec = w_v[j, pl.ds(g * LANES, LANES)]
                for l in range(LANES):
                    wv = wvec[l]
                    r = g * LANES + l
                    for vg in range(vregs):
                        sl = pl.ds(vg * LANES, LANES)
                        rows[r, sl] = rows[r, sl] * wv
                return 0
            lax.fori_loop(0, CHUNK // LANES, scale_body, 0)

        def run_edges(tile_base, npass):
            def pass_body(p, _):
                base = pl.multiple_of(tile_base + p * NSTAGE, NSTAGE)
                pltpu.sync_copy(src_hbm.at[pl.ds(base, NSTAGE)], src_v)
                pltpu.sync_copy(dst_hbm.at[pl.ds(base, NSTAGE)], dst_v)
                pltpu.sync_copy(w_hbm.at[pl.ds(base, NSTAGE)], w_v)
                pltpu.async_copy(tbl_hbm.at[src_v.at[0]], rows0, sem0)

                def pair_body(i2, _):
                    j0 = 2 * i2
                    j1 = j0 + 1
                    j2 = jnp.minimum(j0 + 2, NSTAGE - 1)
                    pltpu.make_async_copy(tbl_hbm.at[src_v.at[j0]], rows0, sem0).wait()
                    pltpu.async_copy(tbl_hbm.at[src_v.at[j1]], rows1, sem1)
                    scale(rows0, j0)
                    pltpu.sync_copy(rows0, acc_sh.at[dst_v.at[j0]], add=True)
                    pltpu.make_async_copy(tbl_hbm.at[src_v.at[j1]], rows1, sem1).wait()
                    pltpu.async_copy(tbl_hbm.at[src_v.at[j2]], rows0, sem0)
                    scale(rows1, j1)
                    pltpu.sync_copy(rows1, acc_sh.at[dst_v.at[j1]], add=True)
                    return 0
                lax.fori_loop(0, NSTAGE // 2, pair_body, 0)
                pltpu.make_async_copy(tbl_hbm.at[src_v.at[NSTAGE - 1]], rows0,
                                      sem0).wait()
                return 0
            lax.fori_loop(0, npass, pass_body, 0)

        @pl.when(c == 0)
        def _():
            run_edges(s * nch0, nch0 // NSTAGE)

        @pl.when(c == 1)
        def _():
            run_edges(NTILES * nch0 + s * nch1, nch1 // NSTAGE)
        plsc.subcore_barrier()
        pltpu.sync_copy(acc_sh.at[pl.ds(s * rpt, rpt)],
                        out_hbm.at[c, pl.ds(s * rpt, rpt)])

    return agg_kernel


def _scale_kernel(d0_ref, d1_ref, x_ref, dinv_ref, xs_ref):
    dv = lax.rsqrt(d0_ref[...] + d1_ref[...] + 1.0)
    dinv_ref[...] = dv
    xs_ref[...] = x_ref[...] * dv


def _mlp_kernel(dinv_ref, p0_ref, p1_ref, x_ref, w1_ref, b1_ref, w2_ref,
                h2_ref, hs2_ref):
    dv = dinv_ref[...]
    t1 = dv * (p0_ref[...] + p1_ref[...]) + dv * dv * x_ref[...]
    h = jnp.dot(t1, w1_ref[...], preferred_element_type=jnp.float32)
    h = jnp.maximum(h + b1_ref[...], 0.0)
    h2 = jnp.dot(h, w2_ref[...], preferred_element_type=jnp.float32)
    h2_ref[...] = h2
    hs2_ref[...] = dv * h2


def _make_out_kernel(n_cls, d2):
    def out_kernel(dinv_ref, q0_ref, q1_ref, h2_ref, b2_ref, out_ref):
        dv = dinv_ref[...]
        z = dv * (q0_ref[...] + q1_ref[...]) + dv * dv * h2_ref[...] + b2_ref[...]
        col = lax.broadcasted_iota(jnp.int32, z.shape, 1)
        mask = col < n_cls
        zm = jnp.where(mask, z, -jnp.inf)
        m = jnp.max(zm, axis=1, keepdims=True)
        ez = jnp.where(mask, jnp.exp(z - m), 0.0)
        lse = jnp.log(jnp.sum(ez, axis=1, keepdims=True))
        out_ref[...] = z - m - lse
    return out_kernel


def kernel(x, edge_index, edge_attr, W1, b1, W2, b2):
    n, d_in = x.shape
    e = edge_index.shape[1]
    d_hid = W1.shape[1]
    n_cls = W2.shape[1]
    d2 = _round_up(n_cls, LANES)
    n_pad = _round_up(n, NTILES * CHUNK)
    ep = _round_up(e, NW * CHUNK * NSTAGE)

    f32 = jnp.float32
    src = jnp.pad(edge_index[0].astype(jnp.int32), (0, ep - e)).reshape(ep // CHUNK, CHUNK)
    dst = jnp.pad(edge_index[1].astype(jnp.int32), (0, ep - e)).reshape(ep // CHUNK, CHUNK)
    w = jnp.pad(edge_attr.astype(f32), (0, ep - e)).reshape(ep // CHUNK, CHUNK)
    xp = jnp.pad(x.astype(f32), ((0, n_pad - n), (0, 0)))
    W2p = jnp.pad(W2.astype(f32), ((0, 0), (0, d2 - n_cls)))
    b2p = jnp.pad(b2.astype(f32), (0, d2 - n_cls)).reshape(1, d2)
    b1r = b1.astype(f32).reshape(1, d_hid)

    deg_p = _make_deg_kernel(ep, n_pad)(dst, w)

    rb = 1024
    grid = (n_pad // rb,)
    dinv, xs = pl.pallas_call(
        _scale_kernel,
        grid=grid,
        in_specs=[
            pl.BlockSpec((rb, 1), lambda i: (i, 0)),
            pl.BlockSpec((rb, 1), lambda i: (i, 0)),
            pl.BlockSpec((rb, d_in), lambda i: (i, 0)),
        ],
        out_specs=[
            pl.BlockSpec((rb, 1), lambda i: (i, 0)),
            pl.BlockSpec((rb, d_in), lambda i: (i, 0)),
        ],
        out_shape=[
            jax.ShapeDtypeStruct((n_pad, 1), f32),
            jax.ShapeDtypeStruct((n_pad, d_in), f32),
        ],
    )(deg_p[0].reshape(n_pad, 1), deg_p[1].reshape(n_pad, 1), xp)

    agg1 = _make_agg_kernel(ep, n_pad, d_in, AGG_SHARE0)(xs, src, dst, w)

    h2, hs2 = pl.pallas_call(
        _mlp_kernel,
        grid=grid,
        in_specs=[
            pl.BlockSpec((rb, 1), lambda i: (i, 0)),
            pl.BlockSpec((rb, d_in), lambda i: (i, 0)),
            pl.BlockSpec((rb, d_in), lambda i: (i, 0)),
            pl.BlockSpec((rb, d_in), lambda i: (i, 0)),
            pl.BlockSpec((d_in, d_hid), lambda i: (0, 0)),
            pl.BlockSpec((1, d_hid), lambda i: (0, 0)),
            pl.BlockSpec((d_hid, d2), lambda i: (0, 0)),
        ],
        out_specs=[
            pl.BlockSpec((rb, d2), lambda i: (i, 0)),
            pl.BlockSpec((rb, d2), lambda i: (i, 0)),
        ],
        out_shape=[
            jax.ShapeDtypeStruct((n_pad, d2), f32),
            jax.ShapeDtypeStruct((n_pad, d2), f32),
        ],
    )(dinv, agg1[0], agg1[1], xp, W1.astype(f32), b1r, W2p)

    agg2 = _make_agg_kernel(ep, n_pad, d2, AGG_SHARE0)(hs2, src, dst, w)

    out = pl.pallas_call(
        _make_out_kernel(n_cls, d2),
        grid=grid,
        in_specs=[
            pl.BlockSpec((rb, 1), lambda i: (i, 0)),
            pl.BlockSpec((rb, d2), lambda i: (i, 0)),
            pl.BlockSpec((rb, d2), lambda i: (i, 0)),
            pl.BlockSpec((rb, d2), lambda i: (i, 0)),
            pl.BlockSpec((1, d2), lambda i: (0, 0)),
        ],
        out_specs=pl.BlockSpec((rb, d2), lambda i: (i, 0)),
        out_shape=jax.ShapeDtypeStruct((n_pad, d2), f32),
    )(dinv, agg2[0], agg2[1], h2, b2p)

    return out[:n, :n_cls]


# speedup vs baseline: 1.0185x; 1.0185x over previous
import functools

import jax
import jax.numpy as jnp
from jax import lax
from jax.experimental import pallas as pl
from jax.experimental.pallas import tpu as pltpu
from jax.experimental.pallas import tpu_sc as plsc

LANES = 16
CHUNK = 128
NSC = 2
NTILES = 16
NW = NSC * NTILES


def _round_up(v, m):
    return (v + m - 1) // m * m


def _sc_mesh():
    return plsc.VectorSubcoreMesh(core_axis_name="c", subcore_axis_name="s")


def _make_deg_kernel(ep, n_pad):
    nch = ep // NW // CHUNK
    rpt = n_pad // NTILES

    @functools.partial(
        pl.kernel,
        out_type=jax.ShapeDtypeStruct((NSC, n_pad), jnp.float32),
        mesh=_sc_mesh(),
        scratch_types=[
            pltpu.VMEM((nch, CHUNK), jnp.int32),
            pltpu.VMEM((nch, CHUNK), jnp.float32),
            pltpu.VMEM((rpt,), jnp.float32),
            pltpu.VMEM_SHARED((n_pad,), jnp.float32),
        ],
    )
    def deg_kernel(dst_hbm, w_hbm, out_hbm, dst_v, w_v, zbuf, acc_sh):
        c = lax.axis_index("c")
        s = lax.axis_index("s")
        wid = c * NTILES + s

        def zero_body(i, _):
            zbuf[pl.ds(i * LANES, LANES)] = jnp.zeros((LANES,), jnp.float32)
            return 0
        lax.fori_loop(0, rpt // LANES, zero_body, 0)
        pltpu.sync_copy(zbuf, acc_sh.at[pl.ds(s * rpt, rpt)])
        pltpu.sync_copy(dst_hbm.at[pl.ds(wid * nch, nch)], dst_v)
        pltpu.sync_copy(w_hbm.at[pl.ds(wid * nch, nch)], w_v)
        plsc.subcore_barrier()

        def chunk_body(j, _):
            pltpu.sync_copy(w_v.at[j], acc_sh.at[dst_v.at[j]], add=True)
            return 0
        lax.fori_loop(0, nch, chunk_body, 0)
        plsc.subcore_barrier()
        pltpu.sync_copy(acc_sh.at[pl.ds(s * rpt, rpt)],
                        out_hbm.at[c, pl.ds(s * rpt, rpt)])

    return deg_kernel


NSTAGE = 16
AGG_SHARE0 = 0.8


def _make_agg_kernel(ep, n_pad, d, share0=0.5):
    tot = ep // NTILES // CHUNK
    nch0 = int(round(tot * share0 / NSTAGE)) * NSTAGE
    nch1 = tot - nch0
    rpt = n_pad // NTILES
    vregs = d // LANES

    @functools.partial(
        pl.kernel,
        out_type=jax.ShapeDtypeStruct((NSC, n_pad, d), jnp.float32),
        mesh=_sc_mesh(),
        scratch_types=[
            pltpu.VMEM((NSTAGE, CHUNK), jnp.int32),
            pltpu.VMEM((NSTAGE, CHUNK), jnp.int32),
            pltpu.VMEM((NSTAGE, CHUNK), jnp.float32),
            pltpu.VMEM((CHUNK, d), jnp.float32),
            pltpu.VMEM((CHUNK, d), jnp.float32),
            pltpu.VMEM_SHARED((n_pad, d), jnp.float32),
            pltpu.SemaphoreType.DMA,
            pltpu.SemaphoreType.DMA,
        ],
        compiler_params=pltpu.CompilerParams(use_tc_tiling_on_sc=False),
    )
    def agg_kernel(tbl_hbm, src_hbm, dst_hbm, w_hbm, out_hbm,
                   src_v, dst_v, w_v, rows0, rows1, acc_sh, sem0, sem1):
        c = lax.axis_index("c")
        s = lax.axis_index("s")
        wid = c * NTILES + s

        def zero_body(i, _):
            r = i // vregs
            g = i % vregs
            rows0[r, pl.ds(g * LANES, LANES)] = jnp.zeros((LANES,), jnp.float32)
            return 0
        lax.fori_loop(0, CHUNK * vregs, zero_body, 0)
        for k in range(rpt // CHUNK):
            pltpu.sync_copy(rows0, acc_sh.at[pl.ds(s * rpt + k * CHUNK, CHUNK)])
        plsc.subcore_barrier()

        def scale(rows, j):
            def scale_body(g, _):
                wvec = w_v[j, pl.ds(g * LANES, LANES)]
                for l in range(LANES):
                    wv = wvec[l]
                    r = g * LANES + l
                    for vg in range(vregs):
                        sl = pl.ds(vg * LANES, LANES)
                        rows[r, sl] = rows[r, sl] * wv
                return 0
            lax.fori_loop(0, CHUNK // LANES, scale_body, 0)

        def run_edges(tile_base, npass):
            def pass_body(p, _):
                base = pl.multiple_of(tile_base + p * NSTAGE, NSTAGE)
                pltpu.sync_copy(src_hbm.at[pl.ds(base, NSTAGE)], src_v)
                pltpu.sync_copy(dst_hbm.at[pl.ds(base, NSTAGE)], dst_v)
                pltpu.sync_copy(w_hbm.at[pl.ds(base, NSTAGE)], w_v)
                pltpu.async_copy(tbl_hbm.at[src_v.at[0]], rows0, sem0)

                def pair_body(i2, _):
                    j0 = 2 * i2
                    j1 = j0 + 1
                    j2 = jnp.minimum(j0 + 2, NSTAGE - 1)
                    pltpu.make_async_copy(tbl_hbm.at[src_v.at[j0]], rows0, sem0).wait()
                    pltpu.async_copy(tbl_hbm.at[src_v.at[j1]], rows1, sem1)
                    scale(rows0, j0)
                    pltpu.sync_copy(rows0, acc_sh.at[dst_v.at[j0]], add=True)
                    pltpu.make_async_copy(tbl_hbm.at[src_v.at[j1]], rows1, sem1).wait()
                    pltpu.async_copy(tbl_hbm.at[src_v.at[j2]], rows0, sem0)
                    scale(rows1, j1)
                    pltpu.sync_copy(rows1, acc_sh.at[dst_v.at[j1]], add=True)
                    return 0
                lax.fori_loop(0, NSTAGE // 2, pair_body, 0)
                pltpu.make_async_copy(tbl_hbm.at[src_v.at[NSTAGE - 1]], rows0,
                                      sem0).wait()
                return 0
            lax.fori_loop(0, npass, pass_body, 0)

        @pl.when(c == 0)
        def _():
            run_edges(s * nch0, nch0 // NSTAGE)

        @pl.when(c == 1)
        def _():
            run_edges(NTILES * nch0 + s * nch1, nch1 // NSTAGE)
        plsc.subcore_barrier()
        pltpu.sync_copy(acc_sh.at[pl.ds(s * rpt, rpt)],
                        out_hbm.at[c, pl.ds(s * rpt, rpt)])

    return agg_kernel


def _scale_kernel(d0_ref, d1_ref, x_ref, dinv_ref, xs_ref):
    dv = lax.rsqrt(d0_ref[...] + d1_ref[...] + 1.0)
    dinv_ref[...] = dv
    xs_ref[...] = x_ref[...] * dv


def _mlp_kernel(dinv_ref, p0_ref, p1_ref, x_ref, w1_ref, b1_ref, w2_ref,
                h2_ref, hs2_ref):
    dv = dinv_ref[...]
    t1 = dv * (p0_ref[...] + p1_ref[...]) + dv * dv * x_ref[...]
    h = jnp.dot(t1, w1_ref[...], preferred_element_type=jnp.float32)
    h = jnp.maximum(h + b1_ref[...], 0.0)
    h2 = jnp.dot(h, w2_ref[...], preferred_element_type=jnp.float32)
    h2_ref[...] = h2
    hs2_ref[...] = dv * h2


def _make_out_kernel(n_cls, d2):
    def out_kernel(dinv_ref, q0_ref, q1_ref, h2_ref, b2_ref, out_ref):
        dv = dinv_ref[...]
        z = dv * (q0_ref[...] + q1_ref[...]) + dv * dv * h2_ref[...] + b2_ref[...]
        col = lax.broadcasted_iota(jnp.int32, z.shape, 1)
        mask = col < n_cls
        zm = jnp.where(mask, z, -jnp.inf)
        m = jnp.max(zm, axis=1, keepdims=True)
        ez = jnp.where(mask, jnp.exp(z - m), 0.0)
        lse = jnp.log(jnp.sum(ez, axis=1, keepdims=True))
        out_ref[...] = z - m - lse
    return out_kernel


def kernel(x, edge_index, edge_attr, W1, b1, W2, b2):
    n, d_in = x.shape
    e = edge_index.shape[1]
    d_hid = W1.shape[1]
    n_cls = W2.shape[1]
    d2 = _round_up(n_cls, LANES)
    n_pad = _round_up(n, NTILES * CHUNK)
    ep = _round_up(e, NW * CHUNK * NSTAGE)

    f32 = jnp.float32
    src = jnp.pad(edge_index[0].astype(jnp.int32), (0, ep - e)).reshape(ep // CHUNK, CHUNK)
    dst = jnp.pad(edge_index[1].astype(jnp.int32), (0, ep - e)).reshape(ep // CHUNK, CHUNK)
    w = jnp.pad(edge_attr.astype(f32), (0, ep - e)).reshape(ep // CHUNK, CHUNK)
    xp = jnp.pad(x.astype(f32), ((0, n_pad - n), (0, 0)))
    W2p = jnp.pad(W2.astype(f32), ((0, 0), (0, d2 - n_cls)))
    b2p = jnp.pad(b2.astype(f32), (0, d2 - n_cls)).reshape(1, d2)
    b1r = b1.astype(f32).reshape(1, d_hid)

    deg_p = _make_deg_kernel(ep, n_pad)(dst, w)

    rb = 1024
    grid = (n_pad // rb,)
    dinv, xs = pl.pallas_call(
        _scale_kernel,
        grid=grid,
        in_specs=[
            pl.BlockSpec((rb, 1), lambda i: (i, 0)),
            pl.BlockSpec((rb, 1), lambda i: (i, 0)),
            pl.BlockSpec((rb, d_in), lambda i: (i, 0)),
        ],
        out_specs=[
            pl.BlockSpec((rb, 1), lambda i: (i, 0)),
            pl.BlockSpec((rb, d_in), lambda i: (i, 0)),
        ],
        out_shape=[
            jax.ShapeDtypeStruct((n_pad, 1), f32),
            jax.ShapeDtypeStruct((n_pad, d_in), f32),
        ],
    )(deg_p[0].reshape(n_pad, 1), deg_p[1].reshape(n_pad, 1), xp)

    agg1 = _make_agg_kernel(ep, n_pad, d_in, AGG_SHARE0)(xs, src, dst, w)

    h2, hs2 = pl.pallas_call(
        _mlp_kernel,
        grid=grid,
        in_specs=[
            pl.BlockSpec((rb, 1), lambda i: (i, 0)),
            pl.BlockSpec((rb, d_in), lambda i: (i, 0)),
            pl.BlockSpec((rb, d_in), lambda i: (i, 0)),
            pl.BlockSpec((rb, d_in), lambda i: (i, 0)),
            pl.BlockSpec((d_in, d_hid), lambda i: (0, 0)),
            pl.BlockSpec((1, d_hid), lambda i: (0, 0)),
            pl.BlockSpec((d_hid, d2), lambda i: (0, 0)),
        ],
        out_specs=[
            pl.BlockSpec((rb, d2), lambda i: (i, 0)),
            pl.BlockSpec((rb, d2), lambda i: (i, 0)),
        ],
        out_shape=[
            jax.ShapeDtypeStruct((n_pad, d2), f32),
            jax.ShapeDtypeStruct((n_pad, d2), f32),
        ],
    )(dinv, agg1[0], agg1[1], xp, W1.astype(f32), b1r, W2p)

    agg2 = _make_agg_kernel(ep, n_pad, d2, AGG_SHARE0)(hs2, src, dst, w)

    out = pl.pallas_call(
        _make_out_kernel(n_cls, d2),
        grid=grid,
        in_specs=[
            pl.BlockSpec((rb, 1), lambda i: (i, 0)),
            pl.BlockSpec((rb, d2), lambda i: (i, 0)),
            pl.BlockSpec((rb, d2), lambda i: (i, 0)),
            pl.BlockSpec((rb, d2), lambda i: (i, 0)),
            pl.BlockSpec((1, d2), lambda i: (0, 0)),
        ],
        out_specs=pl.BlockSpec((rb, d2), lambda i: (i, 0)),
        out_shape=jax.ShapeDtypeStruct((n_pad, d2), f32),
    )(dinv, agg2[0], agg2[1], h2, b2p)

    return out[:n, :n_cls]


# speedup vs baseline: 1.0813x; 1.0617x over previous
import functools

import jax
import jax.numpy as jnp
from jax import lax
from jax.experimental import pallas as pl
from jax.experimental.pallas import tpu as pltpu
from jax.experimental.pallas import tpu_sc as plsc

LANES = 16
CHUNK = 128
NSC = 2
NTILES = 16
NW = NSC * NTILES


def _round_up(v, m):
    return (v + m - 1) // m * m


def _sc_mesh():
    return plsc.VectorSubcoreMesh(core_axis_name="c", subcore_axis_name="s")


def _make_deg_kernel(ep, n_pad):
    nch = ep // NW // CHUNK
    rpt = n_pad // NTILES

    @functools.partial(
        pl.kernel,
        out_type=jax.ShapeDtypeStruct((NSC, n_pad), jnp.float32),
        mesh=_sc_mesh(),
        scratch_types=[
            pltpu.VMEM((nch, CHUNK), jnp.int32),
            pltpu.VMEM((nch, CHUNK), jnp.float32),
            pltpu.VMEM((rpt,), jnp.float32),
            pltpu.VMEM_SHARED((n_pad,), jnp.float32),
        ],
    )
    def deg_kernel(dst_hbm, w_hbm, out_hbm, dst_v, w_v, zbuf, acc_sh):
        c = lax.axis_index("c")
        s = lax.axis_index("s")
        wid = c * NTILES + s

        def zero_body(i, _):
            zbuf[pl.ds(i * LANES, LANES)] = jnp.zeros((LANES,), jnp.float32)
            return 0
        lax.fori_loop(0, rpt // LANES, zero_body, 0)
        pltpu.sync_copy(zbuf, acc_sh.at[pl.ds(s * rpt, rpt)])
        pltpu.sync_copy(dst_hbm.at[pl.ds(wid * nch, nch)], dst_v)
        pltpu.sync_copy(w_hbm.at[pl.ds(wid * nch, nch)], w_v)
        plsc.subcore_barrier()

        def chunk_body(j, _):
            pltpu.sync_copy(w_v.at[j], acc_sh.at[dst_v.at[j]], add=True)
            return 0
        lax.fori_loop(0, nch, chunk_body, 0)
        plsc.subcore_barrier()
        pltpu.sync_copy(acc_sh.at[pl.ds(s * rpt, rpt)],
                        out_hbm.at[c, pl.ds(s * rpt, rpt)])

    return deg_kernel


NSTAGE = 16
AGG_SHARE0 = 0.8


def _make_agg_kernel(ep, n_pad, d, share0=0.5, nbuf=2):
    tot = ep // NTILES // CHUNK
    nch0 = int(round(tot * share0 / NSTAGE)) * NSTAGE
    nch1 = tot - nch0
    rpt = n_pad // NTILES
    vregs = d // LANES

    @functools.partial(
        pl.kernel,
        out_type=jax.ShapeDtypeStruct((NSC, n_pad, d), jnp.float32),
        mesh=_sc_mesh(),
        scratch_types=[
            pltpu.VMEM((NSTAGE, CHUNK), jnp.int32),
            pltpu.VMEM((NSTAGE, CHUNK), jnp.int32),
            pltpu.VMEM((NSTAGE, CHUNK), jnp.float32),
        ] + [pltpu.VMEM((CHUNK, d), jnp.float32) for _ in range(nbuf)]
        + [pltpu.VMEM_SHARED((n_pad, d), jnp.float32)]
        + [pltpu.SemaphoreType.DMA for _ in range(2 * nbuf)],
        compiler_params=pltpu.CompilerParams(use_tc_tiling_on_sc=False),
    )
    def agg_kernel(tbl_hbm, src_hbm, dst_hbm, w_hbm, out_hbm, *bufs):
        src_v, dst_v, w_v = bufs[0], bufs[1], bufs[2]
        rows = list(bufs[3:3 + nbuf])
        acc_sh = bufs[3 + nbuf]
        gsem = list(bufs[4 + nbuf:4 + 2 * nbuf])
        ssem = list(bufs[4 + 2 * nbuf:4 + 3 * nbuf])
        c = lax.axis_index("c")
        s = lax.axis_index("s")
        rows0 = rows[0]

        def zero_body(i, _):
            r = i // vregs
            g = i % vregs
            rows0[r, pl.ds(g * LANES, LANES)] = jnp.zeros((LANES,), jnp.float32)
            return 0
        lax.fori_loop(0, CHUNK * vregs, zero_body, 0)
        for k in range(rpt // CHUNK):
            pltpu.sync_copy(rows0, acc_sh.at[pl.ds(s * rpt + k * CHUNK, CHUNK)])
        plsc.subcore_barrier()

        def scale(rows, j):
            def scale_body(g, _):
                wvec = w_v[j, pl.ds(g * LANES, LANES)]
                for l in range(LANES):
                    wv = wvec[l]
                    r = g * LANES + l
                    for vg in range(vregs):
                        sl = pl.ds(vg * LANES, LANES)
                        rows[r, sl] = rows[r, sl] * wv
                return 0
            lax.fori_loop(0, CHUNK // LANES, scale_body, 0)

        def run_edges(tile_base, npass):
            def pass_body(p, _):
                base = pl.multiple_of(tile_base + p * NSTAGE, NSTAGE)
                pltpu.sync_copy(src_hbm.at[pl.ds(base, NSTAGE)], src_v)
                pltpu.sync_copy(dst_hbm.at[pl.ds(base, NSTAGE)], dst_v)
                pltpu.sync_copy(w_hbm.at[pl.ds(base, NSTAGE)], w_v)
                for b in range(nbuf - 1):
                    pltpu.async_copy(tbl_hbm.at[src_v.at[b]], rows[b], gsem[b])

                def chunk_group(g, _):
                    for b in range(nbuf):
                        k = g * nbuf + b
                        pltpu.make_async_copy(tbl_hbm.at[src_v.at[k]],
                                              rows[b], gsem[b]).wait()
                        scale(rows[b], k)
                        pltpu.async_copy(rows[b], acc_sh.at[dst_v.at[k]],
                                         ssem[b], add=True)
                        b2 = (b + nbuf - 1) % nbuf
                        kp = k + nbuf - 1

                        @pl.when(k >= 1)
                        def _():
                            pltpu.make_async_copy(
                                rows[b2], acc_sh.at[dst_v.at[k - 1]],
                                ssem[b2]).wait()

                        @pl.when(kp < NSTAGE)
                        def _():
                            pltpu.async_copy(tbl_hbm.at[src_v.at[kp]],
                                             rows[b2], gsem[b2])
                    return 0
                lax.fori_loop(0, NSTAGE // nbuf, chunk_group, 0)
                bl = (NSTAGE - 1) % nbuf
                pltpu.make_async_copy(rows[bl], acc_sh.at[dst_v.at[NSTAGE - 1]],
                                      ssem[bl]).wait()
                return 0
            lax.fori_loop(0, npass, pass_body, 0)

        @pl.when(c == 0)
        def _():
            run_edges(s * nch0, nch0 // NSTAGE)

        @pl.when(c == 1)
        def _():
            run_edges(NTILES * nch0 + s * nch1, nch1 // NSTAGE)
        plsc.subcore_barrier()
        pltpu.sync_copy(acc_sh.at[pl.ds(s * rpt, rpt)],
                        out_hbm.at[c, pl.ds(s * rpt, rpt)])

    return agg_kernel


def _scale_kernel(d0_ref, d1_ref, x_ref, dinv_ref, xs_ref):
    dv = lax.rsqrt(d0_ref[...] + d1_ref[...] + 1.0)
    dinv_ref[...] = dv
    xs_ref[...] = x_ref[...] * dv


def _mlp_kernel(dinv_ref, p0_ref, p1_ref, x_ref, w1_ref, b1_ref, w2_ref,
                h2_ref, hs2_ref):
    dv = dinv_ref[...]
    t1 = dv * (p0_ref[...] + p1_ref[...]) + dv * dv * x_ref[...]
    h = jnp.dot(t1, w1_ref[...], preferred_element_type=jnp.float32)
    h = jnp.maximum(h + b1_ref[...], 0.0)
    h2 = jnp.dot(h, w2_ref[...], preferred_element_type=jnp.float32)
    h2_ref[...] = h2
    hs2_ref[...] = dv * h2


def _make_out_kernel(n_cls, d2):
    def out_kernel(dinv_ref, q0_ref, q1_ref, h2_ref, b2_ref, out_ref):
        dv = dinv_ref[...]
        z = dv * (q0_ref[...] + q1_ref[...]) + dv * dv * h2_ref[...] + b2_ref[...]
        col = lax.broadcasted_iota(jnp.int32, z.shape, 1)
        mask = col < n_cls
        zm = jnp.where(mask, z, -jnp.inf)
        m = jnp.max(zm, axis=1, keepdims=True)
        ez = jnp.where(mask, jnp.exp(z - m), 0.0)
        lse = jnp.log(jnp.sum(ez, axis=1, keepdims=True))
        out_ref[...] = z - m - lse
    return out_kernel


def kernel(x, edge_index, edge_attr, W1, b1, W2, b2):
    n, d_in = x.shape
    e = edge_index.shape[1]
    d_hid = W1.shape[1]
    n_cls = W2.shape[1]
    d2 = _round_up(n_cls, LANES)
    n_pad = _round_up(n, NTILES * CHUNK)
    ep = _round_up(e, NW * CHUNK * NSTAGE)

    f32 = jnp.float32
    src = jnp.pad(edge_index[0].astype(jnp.int32), (0, ep - e)).reshape(ep // CHUNK, CHUNK)
    dst = jnp.pad(edge_index[1].astype(jnp.int32), (0, ep - e)).reshape(ep // CHUNK, CHUNK)
    w = jnp.pad(edge_attr.astype(f32), (0, ep - e)).reshape(ep // CHUNK, CHUNK)
    xp = jnp.pad(x.astype(f32), ((0, n_pad - n), (0, 0)))
    W2p = jnp.pad(W2.astype(f32), ((0, 0), (0, d2 - n_cls)))
    b2p = jnp.pad(b2.astype(f32), (0, d2 - n_cls)).reshape(1, d2)
    b1r = b1.astype(f32).reshape(1, d_hid)

    deg_p = _make_deg_kernel(ep, n_pad)(dst, w)

    rb = 1024
    grid = (n_pad // rb,)
    dinv, xs = pl.pallas_call(
        _scale_kernel,
        grid=grid,
        in_specs=[
            pl.BlockSpec((rb, 1), lambda i: (i, 0)),
            pl.BlockSpec((rb, 1), lambda i: (i, 0)),
            pl.BlockSpec((rb, d_in), lambda i: (i, 0)),
        ],
        out_specs=[
            pl.BlockSpec((rb, 1), lambda i: (i, 0)),
            pl.BlockSpec((rb, d_in), lambda i: (i, 0)),
        ],
        out_shape=[
            jax.ShapeDtypeStruct((n_pad, 1), f32),
            jax.ShapeDtypeStruct((n_pad, d_in), f32),
        ],
    )(deg_p[0].reshape(n_pad, 1), deg_p[1].reshape(n_pad, 1), xp)

    agg1 = _make_agg_kernel(ep, n_pad, d_in, AGG_SHARE0, nbuf=2)(xs, src, dst, w)

    h2, hs2 = pl.pallas_call(
        _mlp_kernel,
        grid=grid,
        in_specs=[
            pl.BlockSpec((rb, 1), lambda i: (i, 0)),
            pl.BlockSpec((rb, d_in), lambda i: (i, 0)),
            pl.BlockSpec((rb, d_in), lambda i: (i, 0)),
            pl.BlockSpec((rb, d_in), lambda i: (i, 0)),
            pl.BlockSpec((d_in, d_hid), lambda i: (0, 0)),
            pl.BlockSpec((1, d_hid), lambda i: (0, 0)),
            pl.BlockSpec((d_hid, d2), lambda i: (0, 0)),
        ],
        out_specs=[
            pl.BlockSpec((rb, d2), lambda i: (i, 0)),
            pl.BlockSpec((rb, d2), lambda i: (i, 0)),
        ],
        out_shape=[
            jax.ShapeDtypeStruct((n_pad, d2), f32),
            jax.ShapeDtypeStruct((n_pad, d2), f32),
        ],
    )(dinv, agg1[0], agg1[1], xp, W1.astype(f32), b1r, W2p)

    agg2 = _make_agg_kernel(ep, n_pad, d2, AGG_SHARE0, nbuf=4)(hs2, src, dst, w)

    out = pl.pallas_call(
        _make_out_kernel(n_cls, d2),
        grid=grid,
        in_specs=[
            pl.BlockSpec((rb, 1), lambda i: (i, 0)),
            pl.BlockSpec((rb, d2), lambda i: (i, 0)),
            pl.BlockSpec((rb, d2), lambda i: (i, 0)),
            pl.BlockSpec((rb, d2), lambda i: (i, 0)),
            pl.BlockSpec((1, d2), lambda i: (0, 0)),
        ],
        out_specs=pl.BlockSpec((rb, d2), lambda i: (i, 0)),
        out_shape=jax.ShapeDtypeStruct((n_pad, d2), f32),
    )(dinv, agg2[0], agg2[1], h2, b2p)

    return out[:n, :n_cls]
